# baseline (speedup 1.0000x reference)
import functools

import jax
import jax.numpy as jnp
from jax import lax
from jax.experimental import pallas as pl
from jax.experimental.pallas import tpu as pltpu
from jax.experimental.pallas import tpu_sc as plsc

_HASH_SIZE = 1048576
_DIMENSIONS = 64
_N_QUERIES = 16384
_PRIME1 = 73856093
_PRIME2 = 19349663

_NC = 2
_NS = 16
_NW = _NC * _NS
_L = 16
_CB = 128
_NCB = _N_QUERIES // _CB
_CPW = _NCB // _NW
_BPW = _CPW * _CB
_TDIM = _DIMENSIONS // 8
_FLAT = _HASH_SIZE * _DIMENSIONS


@functools.partial(
    pl.kernel,
    out_type=jax.ShapeDtypeStruct((_N_QUERIES * _DIMENSIONS,), jnp.float32),
    mesh=plsc.VectorSubcoreMesh(core_axis_name="c", subcore_axis_name="s"),
    compiler_params=pltpu.CompilerParams(use_tc_tiling_on_sc=False),
    scratch_types=[
        pltpu.VMEM((_CPW, 2, _CB), jnp.float32),
        pltpu.VMEM((_CPW * _CB // 2, _CB), jnp.int32),
        pltpu.VMEM((_BPW * _DIMENSIONS,), jnp.float32),
        [pltpu.SemaphoreType.DMA] * _CPW,
        pltpu.SemaphoreType.DMA,
    ],
)
def _hash_gather(p3_hbm, gflat_hbm, out_hbm, pos_v, idx_v, rows_v, gsems, wsem):
    wid = lax.axis_index("s") * _NC + lax.axis_index("c")

    pltpu.sync_copy(p3_hbm.at[pl.ds(wid * _CPW, _CPW)], pos_v)

    cvecs = []
    for k in range(4):
        dv = lax.iota(jnp.int32, _L) + k * _L
        cvecs.append((dv >> 3) * (_HASH_SIZE * 8) + (dv & 7) * _CB)

    gathers = [[] for _ in range(_CPW)]
    for cb in range(_CPW):
        def body(q16, carry, cb=cb):
            ix = pos_v[cb, 0, pl.ds(q16 * _L, _L)].astype(jnp.int32)
            iy = pos_v[cb, 1, pl.ds(q16 * _L, _L)].astype(jnp.int32)
            h = ((ix * _PRIME1) ^ (iy * _PRIME2)) & (_HASH_SIZE - 1)
            hterm = (h >> 7) * 1024 + (h & 127)
            for ql in range(_L):
                bcast = hterm.at[jnp.full((_L,), ql, jnp.int32)].get(
                    mode="promise_in_bounds"
                )
                stream = cb * 64 + q16 * 8 + ql // 2
                lane0 = (ql % 2) * 64
                for k in range(4):
                    idx_v[stream, pl.ds(lane0 + k * _L, _L)] = bcast + cvecs[k]
            return carry

        lax.fori_loop(0, _CB // _L, body, 0)
        for j in range(64):
            stream = cb * 64 + j
            c = pltpu.make_async_copy(
                gflat_hbm.at[idx_v.at[stream]],
                rows_v.at[pl.ds(stream * _CB, _CB)],
                gsems[cb],
            )
            c.start()
            gathers[cb].append(c)

    writebacks = []
    for cb in range(_CPW):
        for c in gathers[cb]:
            c.wait()
        w = pltpu.make_async_copy(
            rows_v.at[pl.ds(cb * _CB * _DIMENSIONS, _CB * _DIMENSIONS)],
            out_hbm.at[
                pl.ds((wid * _CPW + cb) * _CB * _DIMENSIONS, _CB * _DIMENSIONS)
            ],
            wsem,
        )
        w.start()
        writebacks.append(w)
    for w in writebacks:
        w.wait()


def kernel(positions, grid):
    gflat = grid.T.reshape(_TDIM, 8, _HASH_SIZE // _CB, _CB)
    gflat = gflat.transpose(0, 2, 1, 3).reshape(_FLAT)
    p3 = positions.T.reshape(2, _NCB, _CB).transpose(1, 0, 2)
    out = _hash_gather(p3, gflat)
    return out.reshape(_N_QUERIES, _DIMENSIONS)


# speedup vs baseline: 1.2226x; 1.2226x over previous
import functools

import jax
import jax.numpy as jnp
from jax import lax
from jax.experimental import pallas as pl
from jax.experimental.pallas import tpu as pltpu
from jax.experimental.pallas import tpu_sc as plsc

_HASH_SIZE = 1048576
_DIMENSIONS = 64
_N_QUERIES = 16384
_PRIME1 = 73856093
_PRIME2 = 19349663

_NC = 2
_NS = 16
_NW = _NC * _NS
_L = 16
_CB = 128
_NCB = _N_QUERIES // _CB
_CPW = _NCB // _NW
_BPW = _CPW * _CB
_TDIM = _DIMENSIONS // 8
_FLAT = _HASH_SIZE * _DIMENSIONS


@functools.partial(
    pl.kernel,
    out_type=jax.ShapeDtypeStruct((_TDIM, _NCB, 8, _CB), jnp.float32),
    mesh=plsc.VectorSubcoreMesh(core_axis_name="c", subcore_axis_name="s"),
    compiler_params=pltpu.CompilerParams(use_tc_tiling_on_sc=False),
    scratch_types=[
        pltpu.VMEM((_CPW, 2, _CB), jnp.float32),
        pltpu.VMEM((_CPW, _CB), jnp.int32),
        pltpu.VMEM((_TDIM, _CPW, 8, _CB), jnp.float32),
        [pltpu.SemaphoreType.DMA] * _CPW,
        pltpu.SemaphoreType.DMA,
    ],
)
def _hash_gather(p3_hbm, gflat_hbm, out_hbm, pos_v, idx_v, rows_v, gsems, wsem):
    wid = lax.axis_index("s") * _NC + lax.axis_index("c")

    pltpu.sync_copy(p3_hbm.at[pl.ds(wid * _CPW, _CPW)], pos_v)

    gathers = [[] for _ in range(_CPW)]
    for cb in range(_CPW):
        for k in range(_CB // _L):
            off = k * _L
            ix = pos_v[cb, 0, pl.ds(off, _L)].astype(jnp.int32)
            iy = pos_v[cb, 1, pl.ds(off, _L)].astype(jnp.int32)
            h = ((ix * _PRIME1) ^ (iy * _PRIME2)) & (_HASH_SIZE - 1)
            idx_v[cb, pl.ds(off, _L)] = (h >> 7) * 1024 + (h & 127)
        for s in range(8):
            for t in range(_TDIM):
                const = t * (_HASH_SIZE * 8) + s * _CB
                src = gflat_hbm.at[pl.ds(const, _FLAT - const)].at[idx_v.at[cb]]
                c = pltpu.make_async_copy(src, rows_v.at[t, cb, s], gsems[cb])
                c.start()
                gathers[cb].append(c)

    writebacks = []
    for cb in range(_CPW):
        for c in gathers[cb]:
            c.wait()
        for t in range(_TDIM):
            w = pltpu.make_async_copy(
                rows_v.at[t, cb], out_hbm.at[t, wid * _CPW + cb], wsem
            )
            w.start()
            writebacks.append(w)
    for w in writebacks:
        w.wait()


def kernel(positions, grid):
    gflat = grid.T.reshape(_TDIM, 8, _HASH_SIZE // _CB, _CB)
    gflat = gflat.transpose(0, 2, 1, 3).reshape(_FLAT)
    p3 = positions.T.reshape(2, _NCB, _CB).transpose(1, 0, 2)
    out4 = _hash_gather(p3, gflat)
    return out4.transpose(0, 2, 1, 3).reshape(_DIMENSIONS, _N_QUERIES).T


# speedup vs baseline: 1.2719x; 1.0404x over previous
import functools

import jax
import jax.numpy as jnp
from jax import lax
from jax.experimental import pallas as pl
from jax.experimental.pallas import tpu as pltpu
from jax.experimental.pallas import tpu_sc as plsc

_HASH_SIZE = 1048576
_DIMENSIONS = 64
_N_QUERIES = 16384
_PRIME1 = 73856093
_PRIME2 = 19349663

_NC = 2
_NS = 16
_NW = _NC * _NS
_L = 16
_CB = 128
_NCB = _N_QUERIES // _CB
_CPW = _NCB // _NW
_BPW = _CPW * _CB
_TDIM = _DIMENSIONS // 8
_FLAT = _HASH_SIZE * _DIMENSIONS


@functools.partial(
    pl.kernel,
    out_type=jax.ShapeDtypeStruct((_TDIM, _NCB, 8 * _CB), jnp.float32),
    mesh=plsc.VectorSubcoreMesh(core_axis_name="c", subcore_axis_name="s"),
    compiler_params=pltpu.CompilerParams(use_tc_tiling_on_sc=False),
    scratch_types=[
        pltpu.VMEM((_CPW, 2, _CB), jnp.float32),
        pltpu.VMEM((_CPW, 2 * _CB), jnp.int32),
        pltpu.VMEM((_TDIM, _CPW, 8 * _CB), jnp.float32),
        [pltpu.SemaphoreType.DMA] * _CPW,
        pltpu.SemaphoreType.DMA,
    ],
)
def _hash_gather(p3_hbm, gflat_hbm, out_hbm, pos_v, idx_v, rows_v, gsems, wsem):
    wid = lax.axis_index("s") * _NC + lax.axis_index("c")

    pltpu.sync_copy(p3_hbm.at[pl.ds(wid * _CPW, _CPW)], pos_v)

    gathers = [[] for _ in range(_CPW)]
    for cb in range(_CPW):
        for k in range(_CB // _L):
            off = k * _L
            ix = pos_v[cb, 0, pl.ds(off, _L)].astype(jnp.int32)
            iy = pos_v[cb, 1, pl.ds(off, _L)].astype(jnp.int32)
            h = ((ix * _PRIME1) ^ (iy * _PRIME2)) & (_HASH_SIZE - 1)
            hterm = (h >> 7) * 1024 + (h & 127)
            idx_v[cb, pl.ds(off, _L)] = hterm
            idx_v[cb, pl.ds(_CB + off, _L)] = hterm + _CB
        for sp in range(4):
            for t in range(_TDIM):
                const = t * (_HASH_SIZE * 8) + sp * (2 * _CB)
                src = gflat_hbm.at[pl.ds(const, _FLAT - const)].at[idx_v.at[cb]]
                c = pltpu.make_async_copy(
                    src, rows_v.at[t, cb, pl.ds(sp * 2 * _CB, 2 * _CB)], gsems[cb]
                )
                c.start()
                gathers[cb].append(c)

    writebacks = []
    for cb in range(_CPW):
        for c in gathers[cb]:
            c.wait()
        for t in range(_TDIM):
            w = pltpu.make_async_copy(
                rows_v.at[t, cb], out_hbm.at[t, wid * _CPW + cb], wsem
            )
            w.start()
            writebacks.append(w)
    for w in writebacks:
        w.wait()


def kernel(positions, grid):
    gflat = grid.T.reshape(_TDIM, 8, _HASH_SIZE // _CB, _CB)
    gflat = gflat.transpose(0, 2, 1, 3).reshape(_FLAT)
    p3 = positions.T.reshape(2, _NCB, _CB).transpose(1, 0, 2)
    out4 = _hash_gather(p3, gflat).reshape(_TDIM, _NCB, 8, _CB)
    return out4.transpose(0, 2, 1, 3).reshape(_DIMENSIONS, _N_QUERIES).T


# speedup vs baseline: 1.2988x; 1.0211x over previous
import functools

import jax
import jax.numpy as jnp
from jax import lax
from jax.experimental import pallas as pl
from jax.experimental.pallas import tpu as pltpu
from jax.experimental.pallas import tpu_sc as plsc

_HASH_SIZE = 1048576
_DIMENSIONS = 64
_N_QUERIES = 16384
_PRIME1 = 73856093
_PRIME2 = 19349663

_NC = 2
_NS = 16
_NW = _NC * _NS
_L = 16
_CB = 128
_NCB = _N_QUERIES // _CB
_CPW = _NCB // _NW
_BPW = _CPW * _CB
_TDIM = _DIMENSIONS // 8
_FLAT = _HASH_SIZE * _DIMENSIONS


@functools.partial(
    pl.kernel,
    out_type=jax.ShapeDtypeStruct((_TDIM, _NCB, 8 * _CB), jnp.float32),
    mesh=plsc.VectorSubcoreMesh(core_axis_name="c", subcore_axis_name="s"),
    compiler_params=pltpu.CompilerParams(use_tc_tiling_on_sc=False),
    scratch_types=[
        pltpu.VMEM((_CPW, 2, _CB), jnp.float32),
        pltpu.VMEM((_CPW, 4 * _CB), jnp.int32),
        pltpu.VMEM((_TDIM, _CPW, 8 * _CB), jnp.float32),
        [pltpu.SemaphoreType.DMA] * _CPW,
        pltpu.SemaphoreType.DMA,
    ],
)
def _hash_gather(p3_hbm, gflat_hbm, out_hbm, pos_v, idx_v, rows_v, gsems, wsem):
    wid = lax.axis_index("s") * _NC + lax.axis_index("c")

    pltpu.sync_copy(p3_hbm.at[pl.ds(wid * _CPW, _CPW)], pos_v)

    gathers = [[] for _ in range(_CPW)]
    for cb in range(_CPW):
        for k in range(_CB // _L):
            off = k * _L
            ix = pos_v[cb, 0, pl.ds(off, _L)].astype(jnp.int32)
            iy = pos_v[cb, 1, pl.ds(off, _L)].astype(jnp.int32)
            h = ((ix * _PRIME1) ^ (iy * _PRIME2)) & (_HASH_SIZE - 1)
            hterm = (h >> 7) * 1024 + (h & 127)
            for sq in range(4):
                idx_v[cb, pl.ds(sq * _CB + off, _L)] = hterm + sq * _CB
        for sp in range(2):
            for t in range(_TDIM):
                const = t * (_HASH_SIZE * 8) + sp * (4 * _CB)
                src = gflat_hbm.at[pl.ds(const, _FLAT - const)].at[idx_v.at[cb]]
                c = pltpu.make_async_copy(
                    src, rows_v.at[t, cb, pl.ds(sp * 4 * _CB, 4 * _CB)], gsems[cb]
                )
                c.start()
                gathers[cb].append(c)

    writebacks = []
    for cb in range(_CPW):
        for c in gathers[cb]:
            c.wait()
        for t in range(_TDIM):
            w = pltpu.make_async_copy(
                rows_v.at[t, cb], out_hbm.at[t, wid * _CPW + cb], wsem
            )
            w.start()
            writebacks.append(w)
    for w in writebacks:
        w.wait()


def kernel(positions, grid):
    gflat = grid.T.reshape(_TDIM, 8, _HASH_SIZE // _CB, _CB)
    gflat = gflat.transpose(0, 2, 1, 3).reshape(_FLAT)
    p3 = positions.T.reshape(2, _NCB, _CB).transpose(1, 0, 2)
    out4 = _hash_gather(p3, gflat).reshape(_TDIM, _NCB, 8, _CB)
    return out4.transpose(0, 2, 1, 3).reshape(_DIMENSIONS, _N_QUERIES).T


# speedup vs baseline: 1.3050x; 1.0047x over previous
import functools

import jax
import jax.numpy as jnp
from jax import lax
from jax.experimental import pallas as pl
from jax.experimental.pallas import tpu as pltpu
from jax.experimental.pallas import tpu_sc as plsc

_HASH_SIZE = 1048576
_DIMENSIONS = 64
_N_QUERIES = 16384
_PRIME1 = 73856093
_PRIME2 = 19349663

_NC = 2
_NS = 16
_NW = _NC * _NS
_L = 16
_CB = 128
_NCB = _N_QUERIES // _CB
_CPW = _NCB // _NW
_BPW = _CPW * _CB
_TDIM = _DIMENSIONS // 8
_FLAT = _HASH_SIZE * _DIMENSIONS


@functools.partial(
    pl.kernel,
    out_type=jax.ShapeDtypeStruct((_TDIM, _NCB, 8 * _CB), jnp.float32),
    mesh=plsc.VectorSubcoreMesh(core_axis_name="c", subcore_axis_name="s"),
    compiler_params=pltpu.CompilerParams(use_tc_tiling_on_sc=False),
    scratch_types=[
        pltpu.VMEM((_CPW, 2, _CB), jnp.float32),
        pltpu.VMEM((_CPW, 8 * _CB), jnp.int32),
        pltpu.VMEM((_TDIM, _CPW, 8 * _CB), jnp.float32),
        [pltpu.SemaphoreType.DMA] * _CPW,
        pltpu.SemaphoreType.DMA,
    ],
)
def _hash_gather(p3_hbm, gflat_hbm, out_hbm, pos_v, idx_v, rows_v, gsems, wsem):
    wid = lax.axis_index("s") * _NC + lax.axis_index("c")

    pltpu.sync_copy(p3_hbm.at[pl.ds(wid * _CPW, _CPW)], pos_v)

    gathers = [[] for _ in range(_CPW)]
    for cb in range(_CPW):
        for k in range(_CB // _L):
            off = k * _L
            ix = pos_v[cb, 0, pl.ds(off, _L)].astype(jnp.int32)
            iy = pos_v[cb, 1, pl.ds(off, _L)].astype(jnp.int32)
            h = ((ix * _PRIME1) ^ (iy * _PRIME2)) & (_HASH_SIZE - 1)
            hterm = (h >> 7) * 1024 + (h & 127)
            for sq in range(8):
                idx_v[cb, pl.ds(sq * _CB + off, _L)] = hterm + sq * _CB
        for t in range(_TDIM):
            const = t * (_HASH_SIZE * 8)
            src = gflat_hbm.at[pl.ds(const, _FLAT - const)].at[idx_v.at[cb]]
            c = pltpu.make_async_copy(src, rows_v.at[t, cb], gsems[cb])
            c.start()
            gathers[cb].append(c)

    writebacks = []
    for cb in range(_CPW):
        for c in gathers[cb]:
            c.wait()
        for t in range(_TDIM):
            w = pltpu.make_async_copy(
                rows_v.at[t, cb], out_hbm.at[t, wid * _CPW + cb], wsem
            )
            w.start()
            writebacks.append(w)
    for w in writebacks:
        w.wait()


def kernel(positions, grid):
    gflat = grid.T.reshape(_TDIM, 8, _HASH_SIZE // _CB, _CB)
    gflat = gflat.transpose(0, 2, 1, 3).reshape(_FLAT)
    p3 = positions.T.reshape(2, _NCB, _CB).transpose(1, 0, 2)
    out4 = _hash_gather(p3, gflat).reshape(_TDIM, _NCB, 8, _CB)
    return out4.transpose(0, 2, 1, 3).reshape(_DIMENSIONS, _N_QUERIES).T


# speedup vs baseline: 1.3249x; 1.0152x over previous
import functools

import jax
import jax.numpy as jnp
from jax import lax
from jax.experimental import pallas as pl
from jax.experimental.pallas import tpu as pltpu
from jax.experimental.pallas import tpu_sc as plsc

_HASH_SIZE = 1048576
_DIMENSIONS = 64
_N_QUERIES = 16384
_PRIME1 = 73856093
_PRIME2 = 19349663

_NC = 2
_NS = 16
_NW = _NC * _NS
_L = 16
_CB = 128
_NCB = _N_QUERIES // _CB
_CPW = _NCB // _NW
_BPW = _CPW * _CB
_TDIM = _DIMENSIONS // 8
_FLAT = _HASH_SIZE * _DIMENSIONS


@functools.partial(
    pl.kernel,
    out_type=jax.ShapeDtypeStruct((_TDIM, _NW, _BPW * 8), jnp.float32),
    mesh=plsc.VectorSubcoreMesh(core_axis_name="c", subcore_axis_name="s"),
    compiler_params=pltpu.CompilerParams(use_tc_tiling_on_sc=False),
    scratch_types=[
        pltpu.VMEM((_CPW, 2, _CB), jnp.float32),
        pltpu.VMEM((_BPW * 8,), jnp.int32),
        pltpu.VMEM((_TDIM, _BPW * 8), jnp.float32),
        [pltpu.SemaphoreType.DMA] * _TDIM,
        pltpu.SemaphoreType.DMA,
    ],
)
def _hash_gather(p3_hbm, gflat_hbm, out_hbm, pos_v, idx_v, rows_v, gsems, wsem):
    wid = lax.axis_index("s") * _NC + lax.axis_index("c")

    pltpu.sync_copy(p3_hbm.at[pl.ds(wid * _CPW, _CPW)], pos_v)

    for cb in range(_CPW):
        for k in range(_CB // _L):
            off = k * _L
            ix = pos_v[cb, 0, pl.ds(off, _L)].astype(jnp.int32)
            iy = pos_v[cb, 1, pl.ds(off, _L)].astype(jnp.int32)
            h = ((ix * _PRIME1) ^ (iy * _PRIME2)) & (_HASH_SIZE - 1)
            hterm = (h >> 7) * 1024 + (h & 127)
            for sq in range(8):
                idx_v[pl.ds(cb * 8 * _CB + sq * _CB + off, _L)] = hterm + sq * _CB

    gathers = []
    for t in range(_TDIM):
        const = t * (_HASH_SIZE * 8)
        src = gflat_hbm.at[pl.ds(const, _FLAT - const)].at[idx_v]
        c = pltpu.make_async_copy(src, rows_v.at[t], gsems[t])
        c.start()
        gathers.append(c)

    writebacks = []
    for t in range(_TDIM):
        gathers[t].wait()
        w = pltpu.make_async_copy(rows_v.at[t], out_hbm.at[t, wid], wsem)
        w.start()
        writebacks.append(w)
    for w in writebacks:
        w.wait()


def kernel(positions, grid):
    gflat = grid.T.reshape(_TDIM, 8, _HASH_SIZE // _CB, _CB)
    gflat = gflat.transpose(0, 2, 1, 3).reshape(_FLAT)
    p3 = positions.T.reshape(2, _NCB, _CB).transpose(1, 0, 2)
    out4 = _hash_gather(p3, gflat).reshape(_TDIM, _NCB, 8, _CB)
    return out4.transpose(0, 2, 1, 3).reshape(_DIMENSIONS, _N_QUERIES).T
